# baseline (speedup 1.0000x reference)
import numpy as np
import jax
import jax.numpy as jnp
from jax.experimental import pallas as pl
from jax.experimental.pallas import tpu as pltpu

_BB = 32
_MB = 512


def _conv1_band() -> np.ndarray:
    R = np.full((480, 28), 75, dtype=np.int32)
    for i in range(5):
        for w in range(32):
            for c in range(3):
                k = i * 96 + w * 3 + c
                for dx in range(2):
                    for kk in range(14):
                        j = w - (2 * kk + dx)
                        if 0 <= j < 5:
                            R[k, dx * 14 + kk] = (i * 5 + j) * 3 + c
    return R


def _conv2_band() -> np.ndarray:
    R = np.full((4480, 10), 1600, dtype=np.int32)
    for i2 in range(5):
        for w in range(14):
            for c in range(64):
                k = i2 * 896 + w * 64 + c
                for fx in range(2):
                    for qw in range(5):
                        j2 = w - (2 * qw + fx)
                        if 0 <= j2 < 5:
                            R[k, fx * 5 + qw] = (i2 * 5 + j2) * 64 + c
    return R


_R1 = _conv1_band()
_R2 = _conv2_band()


def _conv_chain(X, w1s, b1, w2s, b2):
    bb = X.shape[1]
    w1a, w1b_, w1c = w1s
    w2a, w2b_ = w2s
    Xr = [X[i:i + 28].reshape(28 * bb, 96) for i in range(5)]
    XcA = jnp.concatenate(
        [r[:, c * 32:c * 32 + 16] for r in Xr for c in range(3)], axis=-1)
    XcB = jnp.concatenate(
        [r[:, c * 32 + 12:c * 32 + 28] for r in Xr for c in range(3)], axis=-1)
    XcC = jnp.concatenate(
        [r[:, c * 32 + 24:c * 32 + 32] for r in Xr for c in range(3)], axis=-1)
    YA = jnp.dot(XcA, w1a, preferred_element_type=jnp.float32)
    YB = jnp.dot(XcB, w1b_, preferred_element_type=jnp.float32)
    YC = jnp.dot(XcC, w1c, preferred_element_type=jnp.float32)
    Y = jnp.concatenate([YA[:, 0:384], YB[:, 0:384], YC[:, 0:128],
                         YA[:, 384:768], YB[:, 384:768], YC[:, 128:256]],
                        axis=-1)
    Y = jnp.maximum(Y[:, 0:896], Y[:, 896:1792])
    Y = jnp.maximum(Y + b1, 0.0).astype(jnp.bfloat16)
    P1 = Y.reshape(14, 2, bb, 896).max(axis=1)
    P1 = P1.reshape(14 * bb, 896)

    Pr = [P1[i * bb:(i + 10) * bb] for i in range(5)]
    PcA = jnp.concatenate([r[:, 0:640] for r in Pr], axis=-1)
    PcB = jnp.concatenate([r[:, 384:896] for r in Pr], axis=-1)
    ZA = jnp.dot(PcA, w2a, preferred_element_type=jnp.float32)
    ZB = jnp.dot(PcB, w2b_, preferred_element_type=jnp.float32)
    Z = jnp.concatenate([ZA[:, 0:192], ZB[:, 0:128],
                         ZA[:, 192:384], ZB[:, 128:256]], axis=-1)
    Z = jnp.maximum(Z[:, 0:320], Z[:, 320:640])
    Z = jnp.maximum(Z + b2, 0.0)
    Zp = Z.reshape(5, 2, bb, 320).max(axis=1)
    return jnp.concatenate(
        [Zp[q] for q in range(5)], axis=-1).astype(jnp.bfloat16)


def _conv_kernel(x_ref, w1a_ref, w1b_ref, w1c_ref, b1_ref,
                 w2a_ref, w2b_ref, b2_ref, o_ref):
    bb = x_ref.shape[1]
    X = x_ref[...]
    w1s = (w1a_ref[...], w1b_ref[...], w1c_ref[...])
    w2s = (w2a_ref[...], w2b_ref[...])
    b1 = b1_ref[...]
    b2 = b2_ref[...]
    if bb >= 16:
        nch = 4 if bb >= 32 else 2
        bh = bb // nch
        o_ref[...] = jnp.concatenate(
            [_conv_chain(X[:, h * bh:(h + 1) * bh, :], w1s, b1, w2s, b2)
             for h in range(nch)], axis=0)
    else:
        o_ref[...] = _conv_chain(X, w1s, b1, w2s, b2)


def _mlp_kernel(x_ref, l1_ref, b1_ref, l2_ref, b2_ref, o_ref):
    h = jnp.dot(x_ref[...], l1_ref[...], preferred_element_type=jnp.float32)
    h = jnp.maximum(h + b1_ref[...], 0.0).astype(jnp.bfloat16)
    o = jnp.dot(h, l2_ref[...], preferred_element_type=jnp.float32)
    o_ref[...] = jnp.maximum(o + b2_ref[...], 0.0)


def kernel(x, w1, b1, w2, b2, l1, lb1, l2, lb2):
    B = x.shape[0]
    bb = B if B < _BB else _BB
    xq = jnp.transpose(x.astype(jnp.bfloat16), (2, 0, 1, 3)).reshape(32, B, 96)
    w1r = w1[:75].reshape(5, 5, 3, 64).transpose(0, 2, 1, 3)
    w1r = w1r.astype(jnp.bfloat16)

    def band1(ows, width):
        cols = [jnp.pad(w1r, ((0, 0), (0, 0), (o, width - 5 - o), (0, 0)))
                for o in ows]
        return jnp.stack(cols, axis=3).reshape(5 * width * 3, 64 * len(ows))

    w1a = band1([2 * kk for kk in range(6)]
                + [2 * kk + 1 for kk in range(6)], 16)
    w1bb = band1([2 * kk - 12 for kk in range(6, 12)]
                 + [2 * kk - 11 for kk in range(6, 12)], 16)
    w1c = band1([24 - 24, 26 - 24, 25 - 24, 27 - 24], 8)
    w2r = w2.reshape(5, 5, 64, 64).astype(jnp.bfloat16)

    def band2(ows, width):
        cols = [jnp.pad(w2r, ((0, 0), (o, width - 5 - o), (0, 0), (0, 0)))
                for o in ows]
        return jnp.stack(cols, axis=3).reshape(5 * width * 64, 64 * len(ows))

    w2a = band2([0, 2, 4, 1, 3, 5], 10)
    w2bb = band2([0, 2, 1, 3], 8)
    b1t = jnp.tile(b1, (1, 14))
    b2t = jnp.tile(b2, (1, 5))
    feats = pl.pallas_call(
        _conv_kernel,
        out_shape=jax.ShapeDtypeStruct((B, 1600), jnp.bfloat16),
        grid=(B // bb,),
        in_specs=[
            pl.BlockSpec((32, bb, 96), lambda b: (0, b, 0)),
            pl.BlockSpec((240, 768), lambda b: (0, 0)),
            pl.BlockSpec((240, 768), lambda b: (0, 0)),
            pl.BlockSpec((120, 256), lambda b: (0, 0)),
            pl.BlockSpec((1, 896), lambda b: (0, 0)),
            pl.BlockSpec((3200, 384), lambda b: (0, 0)),
            pl.BlockSpec((2560, 256), lambda b: (0, 0)),
            pl.BlockSpec((1, 320), lambda b: (0, 0)),
        ],
        out_specs=pl.BlockSpec((bb, 1600), lambda b: (b, 0)),
        compiler_params=pltpu.CompilerParams(
            dimension_semantics=("parallel",)),
    )(xq, w1a, w1bb, w1c, b1t, w2a, w2bb, b2t)
    mb = B if B < _MB else _MB
    return pl.pallas_call(
        _mlp_kernel,
        out_shape=jax.ShapeDtypeStruct((B, 192), jnp.float32),
        grid=(B // mb,),
        in_specs=[
            pl.BlockSpec((mb, 1600), lambda i: (i, 0)),
            pl.BlockSpec((1600, 384), lambda i: (0, 0)),
            pl.BlockSpec((1, 384), lambda i: (0, 0)),
            pl.BlockSpec((384, 192), lambda i: (0, 0)),
            pl.BlockSpec((1, 192), lambda i: (0, 0)),
        ],
        out_specs=pl.BlockSpec((mb, 192), lambda i: (i, 0)),
        compiler_params=pltpu.CompilerParams(
            dimension_semantics=("parallel",)),
    )(feats, l1.astype(jnp.bfloat16), lb1, l2.astype(jnp.bfloat16), lb2)


# speedup vs baseline: 1.2784x; 1.2784x over previous
import numpy as np
import jax
import jax.numpy as jnp
from jax.experimental import pallas as pl
from jax.experimental.pallas import tpu as pltpu

_BB = 64
_MB = 512


def _conv1_band() -> np.ndarray:
    R = np.full((480, 28), 75, dtype=np.int32)
    for i in range(5):
        for w in range(32):
            for c in range(3):
                k = i * 96 + w * 3 + c
                for dx in range(2):
                    for kk in range(14):
                        j = w - (2 * kk + dx)
                        if 0 <= j < 5:
                            R[k, dx * 14 + kk] = (i * 5 + j) * 3 + c
    return R


def _conv2_band() -> np.ndarray:
    R = np.full((4480, 10), 1600, dtype=np.int32)
    for i2 in range(5):
        for w in range(14):
            for c in range(64):
                k = i2 * 896 + w * 64 + c
                for fx in range(2):
                    for qw in range(5):
                        j2 = w - (2 * qw + fx)
                        if 0 <= j2 < 5:
                            R[k, fx * 5 + qw] = (i2 * 5 + j2) * 64 + c
    return R


_R1 = _conv1_band()
_R2 = _conv2_band()


def _conv_chain(X, w1s, b1, w2s, b2):
    bb = X.shape[1]
    w1a, w1b_, w1c = w1s
    w2a, w2b_ = w2s
    Xr = [X[i:i + 28].reshape(28 * bb, 96) for i in range(5)]
    XcA = jnp.concatenate(
        [r[:, c * 32:c * 32 + 16] for r in Xr for c in range(3)], axis=-1)
    XcB = jnp.concatenate(
        [r[:, c * 32 + 12:c * 32 + 28] for r in Xr for c in range(3)], axis=-1)
    XcC = jnp.concatenate(
        [r[:, c * 32 + 24:c * 32 + 32] for r in Xr for c in range(3)], axis=-1)
    YA = jnp.dot(XcA, w1a, preferred_element_type=jnp.float32)
    YB = jnp.dot(XcB, w1b_, preferred_element_type=jnp.float32)
    YC = jnp.dot(XcC, w1c, preferred_element_type=jnp.float32)
    Y = jnp.concatenate([YA[:, 0:384], YB[:, 0:384], YC[:, 0:128],
                         YA[:, 384:768], YB[:, 384:768], YC[:, 128:256]],
                        axis=-1)
    Y = jnp.maximum(Y[:, 0:896], Y[:, 896:1792])
    Y = jnp.maximum(Y + b1, 0.0).astype(jnp.bfloat16)
    P1 = Y.reshape(14, 2, bb, 896).max(axis=1)
    P1 = P1.reshape(14 * bb, 896)

    Pr = [P1[i * bb:(i + 10) * bb] for i in range(5)]
    PcA = jnp.concatenate([r[:, 0:640] for r in Pr], axis=-1)
    PcB = jnp.concatenate([r[:, 384:896] for r in Pr], axis=-1)
    ZA = jnp.dot(PcA, w2a, preferred_element_type=jnp.float32)
    ZB = jnp.dot(PcB, w2b_, preferred_element_type=jnp.float32)
    Z = jnp.concatenate([ZA[:, 0:192], ZB[:, 0:128],
                         ZA[:, 192:384], ZB[:, 128:256]], axis=-1)
    Z = jnp.maximum(Z[:, 0:320], Z[:, 320:640])
    Z = jnp.maximum(Z + b2, 0.0)
    Zp = Z.reshape(5, 2, bb, 320).max(axis=1)
    return jnp.concatenate(
        [Zp[q] for q in range(5)], axis=-1).astype(jnp.bfloat16)


def _conv_kernel(x_ref, w1a_ref, w1b_ref, w1c_ref, b1_ref,
                 w2a_ref, w2b_ref, b2_ref, o_ref):
    bb = x_ref.shape[1]
    X = x_ref[...]
    w1s = (w1a_ref[...], w1b_ref[...], w1c_ref[...])
    w2s = (w2a_ref[...], w2b_ref[...])
    b1 = b1_ref[...]
    b2 = b2_ref[...]
    if bb >= 16:
        bh = bb // 2
        o_ref[...] = jnp.concatenate(
            [_conv_chain(X[:, h * bh:(h + 1) * bh, :], w1s, b1, w2s, b2)
             for h in range(2)], axis=0)
    else:
        o_ref[...] = _conv_chain(X, w1s, b1, w2s, b2)


def _mlp_kernel(x_ref, l1_ref, b1_ref, l2_ref, b2_ref, o_ref):
    h = jnp.dot(x_ref[...], l1_ref[...], preferred_element_type=jnp.float32)
    h = jnp.maximum(h + b1_ref[...], 0.0).astype(jnp.bfloat16)
    o = jnp.dot(h, l2_ref[...], preferred_element_type=jnp.float32)
    o_ref[...] = jnp.maximum(o + b2_ref[...], 0.0)


def kernel(x, w1, b1, w2, b2, l1, lb1, l2, lb2):
    B = x.shape[0]
    bb = B if B < _BB else _BB
    xq = jnp.transpose(x.astype(jnp.bfloat16), (2, 0, 1, 3)).reshape(32, B, 96)
    w1r = w1[:75].reshape(5, 5, 3, 64).transpose(0, 2, 1, 3)
    w1r = w1r.astype(jnp.bfloat16)

    def band1(ows, width):
        cols = [jnp.pad(w1r, ((0, 0), (0, 0), (o, width - 5 - o), (0, 0)))
                for o in ows]
        return jnp.stack(cols, axis=3).reshape(5 * width * 3, 64 * len(ows))

    w1a = band1([2 * kk for kk in range(6)]
                + [2 * kk + 1 for kk in range(6)], 16)
    w1bb = band1([2 * kk - 12 for kk in range(6, 12)]
                 + [2 * kk - 11 for kk in range(6, 12)], 16)
    w1c = band1([24 - 24, 26 - 24, 25 - 24, 27 - 24], 8)
    w2r = w2.reshape(5, 5, 64, 64).astype(jnp.bfloat16)

    def band2(ows, width):
        cols = [jnp.pad(w2r, ((0, 0), (o, width - 5 - o), (0, 0), (0, 0)))
                for o in ows]
        return jnp.stack(cols, axis=3).reshape(5 * width * 64, 64 * len(ows))

    w2a = band2([0, 2, 4, 1, 3, 5], 10)
    w2bb = band2([0, 2, 1, 3], 8)
    b1t = jnp.tile(b1, (1, 14))
    b2t = jnp.tile(b2, (1, 5))
    feats = pl.pallas_call(
        _conv_kernel,
        out_shape=jax.ShapeDtypeStruct((B, 1600), jnp.bfloat16),
        grid=(B // bb,),
        in_specs=[
            pl.BlockSpec((32, bb, 96), lambda b: (0, b, 0)),
            pl.BlockSpec((240, 768), lambda b: (0, 0)),
            pl.BlockSpec((240, 768), lambda b: (0, 0)),
            pl.BlockSpec((120, 256), lambda b: (0, 0)),
            pl.BlockSpec((1, 896), lambda b: (0, 0)),
            pl.BlockSpec((3200, 384), lambda b: (0, 0)),
            pl.BlockSpec((2560, 256), lambda b: (0, 0)),
            pl.BlockSpec((1, 320), lambda b: (0, 0)),
        ],
        out_specs=pl.BlockSpec((bb, 1600), lambda b: (b, 0)),
        compiler_params=pltpu.CompilerParams(
            dimension_semantics=("parallel",)),
    )(xq, w1a, w1bb, w1c, b1t, w2a, w2bb, b2t)
    mb = B if B < _MB else _MB
    return pl.pallas_call(
        _mlp_kernel,
        out_shape=jax.ShapeDtypeStruct((B, 192), jnp.float32),
        grid=(B // mb,),
        in_specs=[
            pl.BlockSpec((mb, 1600), lambda i: (i, 0)),
            pl.BlockSpec((1600, 384), lambda i: (0, 0)),
            pl.BlockSpec((1, 384), lambda i: (0, 0)),
            pl.BlockSpec((384, 192), lambda i: (0, 0)),
            pl.BlockSpec((1, 192), lambda i: (0, 0)),
        ],
        out_specs=pl.BlockSpec((mb, 192), lambda i: (i, 0)),
        compiler_params=pltpu.CompilerParams(
            dimension_semantics=("parallel",)),
    )(feats, l1.astype(jnp.bfloat16), lb1, l2.astype(jnp.bfloat16), lb2)


# speedup vs baseline: 1.3143x; 1.0281x over previous
import numpy as np
import jax
import jax.numpy as jnp
from jax.experimental import pallas as pl
from jax.experimental.pallas import tpu as pltpu

_BB = 128
_MB = 512


def _conv1_band() -> np.ndarray:
    R = np.full((480, 28), 75, dtype=np.int32)
    for i in range(5):
        for w in range(32):
            for c in range(3):
                k = i * 96 + w * 3 + c
                for dx in range(2):
                    for kk in range(14):
                        j = w - (2 * kk + dx)
                        if 0 <= j < 5:
                            R[k, dx * 14 + kk] = (i * 5 + j) * 3 + c
    return R


def _conv2_band() -> np.ndarray:
    R = np.full((4480, 10), 1600, dtype=np.int32)
    for i2 in range(5):
        for w in range(14):
            for c in range(64):
                k = i2 * 896 + w * 64 + c
                for fx in range(2):
                    for qw in range(5):
                        j2 = w - (2 * qw + fx)
                        if 0 <= j2 < 5:
                            R[k, fx * 5 + qw] = (i2 * 5 + j2) * 64 + c
    return R


_R1 = _conv1_band()
_R2 = _conv2_band()


def _conv_chain(X, w1s, b1, w2s, b2):
    bb = X.shape[1]
    w1a, w1b_, w1c = w1s
    w2a, w2b_ = w2s
    Xr = [X[i:i + 28].reshape(28 * bb, 96) for i in range(5)]
    XcA = jnp.concatenate(
        [r[:, c * 32:c * 32 + 16] for r in Xr for c in range(3)], axis=-1)
    XcB = jnp.concatenate(
        [r[:, c * 32 + 12:c * 32 + 28] for r in Xr for c in range(3)], axis=-1)
    XcC = jnp.concatenate(
        [r[:, c * 32 + 24:c * 32 + 32] for r in Xr for c in range(3)], axis=-1)
    YA = jnp.dot(XcA, w1a, preferred_element_type=jnp.float32)
    YB = jnp.dot(XcB, w1b_, preferred_element_type=jnp.float32)
    YC = jnp.dot(XcC, w1c, preferred_element_type=jnp.float32)
    Y = jnp.concatenate([YA[:, 0:384], YB[:, 0:384], YC[:, 0:128],
                         YA[:, 384:768], YB[:, 384:768], YC[:, 128:256]],
                        axis=-1)
    Y = jnp.maximum(Y[:, 0:896], Y[:, 896:1792])
    Y = jnp.maximum(Y + b1, 0.0).astype(jnp.bfloat16)
    P1 = Y.reshape(14, 2, bb, 896).max(axis=1)
    P1 = P1.reshape(14 * bb, 896)

    Pr = [P1[i * bb:(i + 10) * bb] for i in range(5)]
    PcA = jnp.concatenate([r[:, 0:640] for r in Pr], axis=-1)
    PcB = jnp.concatenate([r[:, 384:896] for r in Pr], axis=-1)
    ZA = jnp.dot(PcA, w2a, preferred_element_type=jnp.float32)
    ZB = jnp.dot(PcB, w2b_, preferred_element_type=jnp.float32)
    Z = jnp.concatenate([ZA[:, 0:192], ZB[:, 0:128],
                         ZA[:, 192:384], ZB[:, 128:256]], axis=-1)
    Z = jnp.maximum(Z[:, 0:320], Z[:, 320:640])
    Z = jnp.maximum(Z + b2, 0.0)
    Zp = Z.reshape(5, 2, bb, 320).max(axis=1)
    return jnp.concatenate(
        [Zp[q] for q in range(5)], axis=-1).astype(jnp.bfloat16)


def _conv_kernel(x_ref, w1a_ref, w1b_ref, w1c_ref, b1_ref,
                 w2a_ref, w2b_ref, b2_ref, o_ref):
    bb = x_ref.shape[1]
    X = x_ref[...]
    w1s = (w1a_ref[...], w1b_ref[...], w1c_ref[...])
    w2s = (w2a_ref[...], w2b_ref[...])
    b1 = b1_ref[...]
    b2 = b2_ref[...]
    if bb >= 16:
        bh = bb // 2
        o_ref[...] = jnp.concatenate(
            [_conv_chain(X[:, h * bh:(h + 1) * bh, :], w1s, b1, w2s, b2)
             for h in range(2)], axis=0)
    else:
        o_ref[...] = _conv_chain(X, w1s, b1, w2s, b2)


def _mlp_kernel(x_ref, l1_ref, b1_ref, l2_ref, b2_ref, o_ref):
    h = jnp.dot(x_ref[...], l1_ref[...], preferred_element_type=jnp.float32)
    h = jnp.maximum(h + b1_ref[...], 0.0).astype(jnp.bfloat16)
    o = jnp.dot(h, l2_ref[...], preferred_element_type=jnp.float32)
    o_ref[...] = jnp.maximum(o + b2_ref[...], 0.0)


def kernel(x, w1, b1, w2, b2, l1, lb1, l2, lb2):
    B = x.shape[0]
    bb = B if B < _BB else _BB
    xq = jnp.transpose(x.astype(jnp.bfloat16), (2, 0, 1, 3)).reshape(32, B, 96)
    w1r = w1[:75].reshape(5, 5, 3, 64).transpose(0, 2, 1, 3)
    w1r = w1r.astype(jnp.bfloat16)

    def band1(ows, width):
        cols = [jnp.pad(w1r, ((0, 0), (0, 0), (o, width - 5 - o), (0, 0)))
                for o in ows]
        return jnp.stack(cols, axis=3).reshape(5 * width * 3, 64 * len(ows))

    w1a = band1([2 * kk for kk in range(6)]
                + [2 * kk + 1 for kk in range(6)], 16)
    w1bb = band1([2 * kk - 12 for kk in range(6, 12)]
                 + [2 * kk - 11 for kk in range(6, 12)], 16)
    w1c = band1([24 - 24, 26 - 24, 25 - 24, 27 - 24], 8)
    w2r = w2.reshape(5, 5, 64, 64).astype(jnp.bfloat16)

    def band2(ows, width):
        cols = [jnp.pad(w2r, ((0, 0), (o, width - 5 - o), (0, 0), (0, 0)))
                for o in ows]
        return jnp.stack(cols, axis=3).reshape(5 * width * 64, 64 * len(ows))

    w2a = band2([0, 2, 4, 1, 3, 5], 10)
    w2bb = band2([0, 2, 1, 3], 8)
    b1t = jnp.tile(b1, (1, 14))
    b2t = jnp.tile(b2, (1, 5))
    feats = pl.pallas_call(
        _conv_kernel,
        out_shape=jax.ShapeDtypeStruct((B, 1600), jnp.bfloat16),
        grid=(B // bb,),
        in_specs=[
            pl.BlockSpec((32, bb, 96), lambda b: (0, b, 0)),
            pl.BlockSpec((240, 768), lambda b: (0, 0)),
            pl.BlockSpec((240, 768), lambda b: (0, 0)),
            pl.BlockSpec((120, 256), lambda b: (0, 0)),
            pl.BlockSpec((1, 896), lambda b: (0, 0)),
            pl.BlockSpec((3200, 384), lambda b: (0, 0)),
            pl.BlockSpec((2560, 256), lambda b: (0, 0)),
            pl.BlockSpec((1, 320), lambda b: (0, 0)),
        ],
        out_specs=pl.BlockSpec((bb, 1600), lambda b: (b, 0)),
        compiler_params=pltpu.CompilerParams(
            dimension_semantics=("parallel",)),
    )(xq, w1a, w1bb, w1c, b1t, w2a, w2bb, b2t)
    mb = B if B < _MB else _MB
    return pl.pallas_call(
        _mlp_kernel,
        out_shape=jax.ShapeDtypeStruct((B, 192), jnp.float32),
        grid=(B // mb,),
        in_specs=[
            pl.BlockSpec((mb, 1600), lambda i: (i, 0)),
            pl.BlockSpec((1600, 384), lambda i: (0, 0)),
            pl.BlockSpec((1, 384), lambda i: (0, 0)),
            pl.BlockSpec((384, 192), lambda i: (0, 0)),
            pl.BlockSpec((1, 192), lambda i: (0, 0)),
        ],
        out_specs=pl.BlockSpec((mb, 192), lambda i: (i, 0)),
        compiler_params=pltpu.CompilerParams(
            dimension_semantics=("parallel",)),
    )(feats, l1.astype(jnp.bfloat16), lb1, l2.astype(jnp.bfloat16), lb2)
